# Initial kernel scaffold; baseline (speedup 1.0000x reference)
import jax
import jax.numpy as jnp
from jax.experimental import pallas as pl


def kernel(t, grid):
    raise NotImplementedError("write your pallas kernel here")


# speedup vs baseline: 3.1217x; 3.1217x over previous
import functools

import jax
import jax.numpy as jnp
import numpy as np
from jax import lax
from jax.experimental import pallas as pl
from jax.experimental.pallas import tpu as pltpu
from jax.experimental.pallas import tpu_sc as plsc

_NT = 1000001
_DT = np.float32(1.0 / (_NT - 1))
_RCP = np.float32(1.0) / _DT
_B = 16384
_D = 32
_NW = 32
_BPW = _B // _NW
_L = 16
_NG = _BPW // _L
_CH = 128
_NCH = (4 * _BPW) // _CH

_mesh = plsc.VectorSubcoreMesh(core_axis_name="c", subcore_axis_name="s",
                               num_cores=2, num_subcores=16)


_GATHER_DNUMS = lax.GatherDimensionNumbers(
    offset_dims=(), collapsed_slice_dims=(0,), start_index_map=(0,))


def _splat(v, lane):
    idx = jnp.full((_L, 1), lane, dtype=jnp.int32)
    return lax.gather(v, idx, _GATHER_DNUMS, slice_sizes=(1,),
                      mode=lax.GatherScatterMode.PROMISE_IN_BOUNDS)


_SCRATCH = [
    pltpu.VMEM((_BPW,), jnp.float32),
    pltpu.VMEM((_NCH, _CH), jnp.int32),
    pltpu.VMEM((4 * _BPW, _D), jnp.float32),
    pltpu.VMEM((_BPW, _D), jnp.float32),
    pltpu.SemaphoreType.DMA,
]


def _body(t_hbm, grid_hbm, out_hbm, t_v, idx_v, rows_v, out_v, sem):
    wid = lax.axis_index("s") * 2 + lax.axis_index("c")
    base = wid * _BPW
    pltpu.sync_copy(t_hbm.at[pl.ds(base, _BPW)], t_v)

    for g in range(_NG):
        tv = t_v[pl.ds(g * _L, _L)]
        m = tv * _RCP + 1.0
        qidx = m.astype(jnp.int32)
        for j in range(4):
            pos = j * _BPW + g * _L
            idx_v[pos // _CH, pl.ds(pos % _CH, _L)] = qidx + (j - 1)

    copies = [
        pltpu.async_copy(grid_hbm.at[idx_v.at[c]],
                         rows_v.at[pl.ds(c * _CH, _CH)], sem)
        for c in range(_NCH)
    ]
    for cp in copies:
        cp.wait()

    def compute_group(g, carry):
        tv = t_v[pl.ds(g * _L, _L)]
        m = tv * _RCP + 1.0
        qidx = m.astype(jnp.int32)
        f = m - qidx.astype(jnp.float32)
        f2 = f * f
        f3 = f2 * f
        c3 = f3 * (1.0 / 6.0)
        c0 = (1.0 / 6.0) - 0.5 * f + 0.5 * f2 - c3
        c1 = (2.0 / 3.0) - f2 + 0.5 * f3
        c2 = (1.0 / 6.0) + 0.5 * f + 0.5 * f2 - 0.5 * f3
        qbase = g * _L
        for qi in range(_L):
            w0 = _splat(c0, qi)
            w1 = _splat(c1, qi)
            w2 = _splat(c2, qi)
            w3 = _splat(c3, qi)
            q = qbase + qi
            for dc in range(_D // _L):
                sl = pl.ds(dc * _L, _L)
                r0 = rows_v[q, sl]
                r1 = rows_v[_BPW + q, sl]
                r2 = rows_v[2 * _BPW + q, sl]
                r3 = rows_v[3 * _BPW + q, sl]
                out_v[q, sl] = w0 * r0 + w1 * r1 + w2 * r2 + w3 * r3
        return carry

    lax.fori_loop(0, _NG, compute_group, 0)
    pltpu.sync_copy(out_v, out_hbm.at[pl.ds(base, _BPW)])


_bspline_sc = pl.kernel(
    _body,
    mesh=_mesh,
    compiler_params=pltpu.CompilerParams(use_tc_tiling_on_sc=False),
    out_type=jax.ShapeDtypeStruct((_B, _D), jnp.float32),
    scratch_types=_SCRATCH,
)


def kernel(t, grid):
    return _bspline_sc(t, grid)
